# baseline (speedup 1.0000x reference)
import functools

import jax
import jax.numpy as jnp
from jax import lax
from jax.experimental import pallas as pl
from jax.experimental.pallas import tpu as pltpu
from jax.experimental.pallas import tpu_sc as plsc

N = 10000
E = 160000
D = 256
H = 256
C = 40

N_PAD = 10240
NCORE = 2
NSUB = 16
NW = NCORE * NSUB
CHUNK = 256
NCHUNK = 20
RPS = N_PAD // NSUB

_f32 = jnp.float32


def _sc_mesh():
    return plsc.VectorSubcoreMesh(core_axis_name="c", subcore_axis_name="s")


@functools.partial(
    pl.kernel,
    out_type=jax.ShapeDtypeStruct((NCORE, N_PAD, 16), _f32),
    mesh=_sc_mesh(),
    compiler_params=pltpu.CompilerParams(use_tc_tiling_on_sc=False),
    scratch_types=[
        pltpu.VMEM((NCHUNK, CHUNK), jnp.int32),
        pltpu.VMEM((CHUNK, 16), _f32),
        pltpu.VMEM_SHARED((N_PAD, 16), _f32),
    ],
)
def _deg_kernel(dst_hbm, ones_hbm, zero_hbm, acc_hbm, dst_v, ones_v, acc_sh):
    c = lax.axis_index("c")
    s = lax.axis_index("s")
    w = c * NSUB + s
    pltpu.sync_copy(dst_hbm.at[w], dst_v)
    pltpu.sync_copy(ones_hbm, ones_v)
    pltpu.sync_copy(zero_hbm, acc_sh.at[pl.ds(s * RPS, RPS)])
    plsc.subcore_barrier()

    def body(j, carry):
        pltpu.sync_copy(ones_v, acc_sh.at[dst_v.at[j]], add=True)
        return carry

    lax.fori_loop(0, NCHUNK, body, 0)
    plsc.subcore_barrier()
    pltpu.sync_copy(acc_sh.at[pl.ds(s * RPS, RPS)],
                    acc_hbm.at[c, pl.ds(s * RPS, RPS)])


def _make_hop(width):
    @functools.partial(
        pl.kernel,
        out_type=jax.ShapeDtypeStruct((NCORE, N_PAD, width), _f32),
        mesh=_sc_mesh(),
        compiler_params=pltpu.CompilerParams(use_tc_tiling_on_sc=False),
        scratch_types=[
            pltpu.VMEM((NCHUNK, CHUNK), jnp.int32),
            pltpu.VMEM((NCHUNK, CHUNK), jnp.int32),
            pltpu.VMEM((CHUNK, width), _f32),
            pltpu.VMEM_SHARED((N_PAD, width), _f32),
            pltpu.SemaphoreType.DMA,
        ],
    )
    def hop(zs_hbm, src_hbm, dst_hbm, zero_hbm, acc_hbm,
            src_v, dst_v, gbuf, acc_sh, sem):
        c = lax.axis_index("c")
        s = lax.axis_index("s")
        w = c * NSUB + s
        pltpu.sync_copy(src_hbm.at[w], src_v)
        pltpu.sync_copy(dst_hbm.at[w], dst_v)
        pltpu.sync_copy(zero_hbm, acc_sh.at[pl.ds(s * RPS, RPS)])
        plsc.subcore_barrier()

        def body(g, carry):
            pltpu.async_copy(zs_hbm.at[src_v.at[g]], gbuf, sem).wait()
            pltpu.sync_copy(gbuf, acc_sh.at[dst_v.at[g]], add=True)
            return carry

        lax.fori_loop(0, NCHUNK, body, 0)
        plsc.subcore_barrier()
        pltpu.sync_copy(acc_sh.at[pl.ds(s * RPS, RPS)],
                        acc_hbm.at[c, pl.ds(s * RPS, RPS)])

    return hop


_hop96 = _make_hop(96)
_hop48 = _make_hop(48)


_BLK = 512


def _dense_body(x_ref, w11, w12, w13, w21, w22, w23, w31, w32, w33,
                b11, b12, b13, b21, b22, b23, b31, b32, b33,
                w1_ref, g2_ref, g3_ref,
                z0a_ref, z0b_ref, u1a_ref, u1b_ref, u2a_ref, u2b_ref,
                u30_ref, y1_ref):
    xb = x_ref[...]
    w1b = [w1_ref[0:H, :], w1_ref[H:2 * H, :], w1_ref[2 * H:3 * H, :]]

    def t(wr, br, j):
        h = jnp.maximum(
            jnp.dot(xb, wr[...], preferred_element_type=_f32) + br[...], 0.0)
        return jnp.dot(h, w1b[j], preferred_element_type=_f32)

    t1 = [t(w11, b11, 0), t(w12, b12, 1), t(w13, b13, 2)]
    t2 = [t(w21, b21, 0), t(w22, b22, 1), t(w23, b23, 2)]
    t3 = [t(w31, b31, 0), t(w32, b32, 1), t(w33, b33, 2)]

    u2 = [g2_ref[0, k] * t2[0] + g2_ref[1, k] * t2[1] + g2_ref[2, k] * t2[2]
          for k in range(3)]
    u3 = [g3_ref[0, k] * t3[0] + g3_ref[1, k] * t3[1] + g3_ref[2, k] * t3[2]
          for k in range(4)]
    y1 = t1[0] + t1[1] + t1[2]

    pad8 = jnp.zeros((xb.shape[0], 8), _f32)

    def p48(a):
        return jnp.concatenate([a, pad8], axis=1)

    z0a_ref[...] = p48(u2[2])
    z0b_ref[...] = p48(u3[3])
    u1a_ref[...] = p48(u2[1])
    u1b_ref[...] = p48(u3[2])
    u2a_ref[...] = p48(u2[0])
    u2b_ref[...] = p48(u3[1])
    u30_ref[...] = p48(u3[0])
    y1_ref[...] = p48(y1)


def _dense_call(x_p, Ws, bs, W1, g2, g3):
    grid = (N_PAD // _BLK,)
    full = lambda shape: pl.BlockSpec(shape, lambda i: (0, 0))
    blk = lambda wdt: pl.BlockSpec((_BLK, wdt), lambda i: (i, 0))
    in_specs = (
        [blk(D)]
        + [full((D, H))] * 9
        + [full((1, H))] * 9
        + [full((3 * H, C))]
        + [pl.BlockSpec(memory_space=pltpu.SMEM)] * 2
    )
    out_specs = [blk(48)] * 8
    out_shape = [jax.ShapeDtypeStruct((N_PAD, 48), _f32)] * 8
    return pl.pallas_call(
        _dense_body, grid=grid, in_specs=in_specs, out_specs=out_specs,
        out_shape=out_shape,
    )(x_p, *Ws, *[b.reshape(1, H) for b in bs], W1, g2, g3)


def _prep_body(z0a_ref, z0b_ref, d0_ref, d1_ref,
               zs0a_ref, zs0b_ref, dinv_ref):
    deg = d0_ref[:, 0:1] + d1_ref[:, 0:1] + 1.0
    dinv48 = jnp.broadcast_to(lax.rsqrt(deg), (z0a_ref.shape[0], 48))
    zs0a_ref[...] = z0a_ref[...] * dinv48
    zs0b_ref[...] = z0b_ref[...] * dinv48
    dinv_ref[...] = dinv48


def _prep(z0a, z0b, deg0, deg1):
    grid = (N_PAD // _BLK2,)
    blk = lambda wdt: pl.BlockSpec((_BLK2, wdt), lambda i: (i, 0))
    return pl.pallas_call(
        _prep_body, grid=grid,
        in_specs=[blk(48), blk(48), blk(16), blk(16)],
        out_specs=[blk(48), blk(48), blk(48)],
        out_shape=[jax.ShapeDtypeStruct((N_PAD, 48), _f32)] * 3,
    )(z0a, z0b, deg0, deg1)


_BLK2 = 1024


def _comb1_body(ua_ref, ub_ref, a0a_ref, a1a_ref, a0b_ref, a1b_ref,
                zsa_ref, zsb_ref, dinv_ref, outa_ref, outb_ref):
    dinv = dinv_ref[...]
    za = ua_ref[...] + dinv * (a0a_ref[...] + a1a_ref[...] + zsa_ref[...])
    zb = ub_ref[...] + dinv * (a0b_ref[...] + a1b_ref[...] + zsb_ref[...])
    outa_ref[...] = za * dinv
    outb_ref[...] = zb * dinv


def _comb2_body(ua_ref, ub_ref, a0a_ref, a1a_ref, a0b_ref, a1b_ref,
                zsa_ref, zsb_ref, dinv_ref, r2_ref, zs3_ref):
    dinv = dinv_ref[...]
    za = ua_ref[...] + dinv * (a0a_ref[...] + a1a_ref[...] + zsa_ref[...])
    zb = ub_ref[...] + dinv * (a0b_ref[...] + a1b_ref[...] + zsb_ref[...])
    r2_ref[...] = za
    zs3_ref[...] = zb * dinv


def _comb(body, UA, UB, acc, zsA, zsB, dinv48):
    grid = (N_PAD // _BLK2,)
    blk = pl.BlockSpec((_BLK2, 48), lambda i: (i, 0))
    return pl.pallas_call(
        body, grid=grid,
        in_specs=[blk] * 9,
        out_specs=[blk, blk],
        out_shape=[jax.ShapeDtypeStruct((N_PAD, 48), _f32),
                   jax.ShapeDtypeStruct((N_PAD, 48), _f32)],
    )(UA, UB, acc[0, :, 0:48], acc[1, :, 0:48], acc[0, :, 48:96],
      acc[1, :, 48:96], zsA, zsB, dinv48)


_BLKF = 1000


def _final_body(y1_ref, r2_ref, u30_ref, a0_ref, a1_ref, zs3_ref, dinv_ref,
                b1_ref, out_ref):
    acc = (a0_ref[...] + a1_ref[...] + zs3_ref[...])[:, 0:C]
    v = (y1_ref[:, 0:C] + r2_ref[:, 0:C] + u30_ref[:, 0:C]
         + dinv_ref[:, 0:C] * acc + b1_ref[...])
    m = jnp.max(v, axis=1, keepdims=True)
    ex = jnp.exp(v - m)
    lse = jnp.log(jnp.sum(ex, axis=1, keepdims=True))
    out_ref[...] = v - m - lse


def _final(y1, r2, u30, a0, a1, zs3, dinv48, b1):
    grid = (N // _BLKF,)
    blk = lambda wdt: pl.BlockSpec((_BLKF, wdt), lambda i: (i, 0))
    return pl.pallas_call(
        _final_body, grid=grid,
        in_specs=[blk(48), blk(48), blk(48), blk(48), blk(48), blk(48),
                  blk(48), pl.BlockSpec((1, C), lambda i: (0, 0))],
        out_specs=blk(C),
        out_shape=jax.ShapeDtypeStruct((N, C), _f32),
    )(y1, r2, u30, a0, a1, zs3, dinv48, b1)


def kernel(x, edge_index, W11, b11, W12, b12, W13, b13, W21, b21, W22, b22,
           W23, b23, W31, b31, W32, b32, W33, b33, W1, b1,
           g21, g22, g23, g31, g32, g33):
    src = edge_index[0]
    dst = edge_index[1]
    flat = NW * NCHUNK * CHUNK
    srcp = jnp.concatenate([src, jnp.zeros((flat - E,), jnp.int32)]
                           ).reshape(NW, NCHUNK, CHUNK)
    dstp = jnp.concatenate([dst, jnp.full((flat - E,), N_PAD - 1, jnp.int32)]
                           ).reshape(NW, NCHUNK, CHUNK)
    x_p = jnp.pad(x, ((0, N_PAD - N), (0, 0)))

    ones16 = jnp.ones((CHUNK, 16), _f32)
    zero16 = jnp.zeros((RPS, 16), _f32)
    zero48 = jnp.zeros((RPS, 48), _f32)
    zero96 = jnp.zeros((RPS, 96), _f32)

    degacc = _deg_kernel(dstp, ones16, zero16)

    g2 = jnp.stack([g21, g22, g23])
    g3 = jnp.stack([g31, g32, g33])
    Ws = (W11, W12, W13, W21, W22, W23, W31, W32, W33)
    bs = (b11, b12, b13, b21, b22, b23, b31, b32, b33)
    z0A, z0B, U1A, U1B, U2A, U2B, u30, y1 = _dense_call(
        x_p, Ws, bs, W1, g2, g3)
    zs0A, zs0B, dinv48 = _prep(z0A, z0B, degacc[0], degacc[1])

    zs0 = jnp.concatenate([zs0A, zs0B], axis=1)
    acc1 = _hop96(zs0, srcp, dstp, zero96)
    zs1A, zs1B = _comb(_comb1_body, U1A, U1B, acc1, zs0A, zs0B, dinv48)
    zs1 = jnp.concatenate([zs1A, zs1B], axis=1)
    acc2 = _hop96(zs1, srcp, dstp, zero96)
    r2, zs3 = _comb(_comb2_body, U2A, U2B, acc2, zs1A, zs1B, dinv48)
    acc3 = _hop48(zs3, srcp, dstp, zero48)

    return _final(y1[:N], r2[:N], u30[:N], acc3[0, :N], acc3[1, :N],
                  zs3[:N], dinv48[:N], b1.reshape(1, C))


# speedup vs baseline: 1.1258x; 1.1258x over previous
import functools

import jax
import jax.numpy as jnp
from jax import lax
from jax.experimental import pallas as pl
from jax.experimental.pallas import tpu as pltpu
from jax.experimental.pallas import tpu_sc as plsc

N = 10000
E = 160000
D = 256
H = 256
C = 40

N_PAD = 10240
NCORE = 2
NSUB = 16
NW = NCORE * NSUB
CHUNK = 256
NCHUNK = 20
RPS = N_PAD // NSUB

_f32 = jnp.float32


def _sc_mesh():
    return plsc.VectorSubcoreMesh(core_axis_name="c", subcore_axis_name="s")


@functools.partial(
    pl.kernel,
    out_type=jax.ShapeDtypeStruct((NCORE, N_PAD, 16), _f32),
    mesh=_sc_mesh(),
    compiler_params=pltpu.CompilerParams(use_tc_tiling_on_sc=False),
    scratch_types=[
        pltpu.VMEM((NCHUNK, CHUNK), jnp.int32),
        pltpu.VMEM((CHUNK, 16), _f32),
        pltpu.VMEM_SHARED((N_PAD, 16), _f32),
    ],
)
def _deg_kernel(dst_hbm, ones_hbm, zero_hbm, acc_hbm, dst_v, ones_v, acc_sh):
    c = lax.axis_index("c")
    s = lax.axis_index("s")
    w = c * NSUB + s
    pltpu.sync_copy(dst_hbm.at[w], dst_v)
    pltpu.sync_copy(ones_hbm, ones_v)
    pltpu.sync_copy(zero_hbm, acc_sh.at[pl.ds(s * RPS, RPS)])
    plsc.subcore_barrier()

    def body(j, carry):
        pltpu.sync_copy(ones_v, acc_sh.at[dst_v.at[j]], add=True)
        return carry

    lax.fori_loop(0, NCHUNK, body, 0)
    plsc.subcore_barrier()
    pltpu.sync_copy(acc_sh.at[pl.ds(s * RPS, RPS)],
                    acc_hbm.at[c, pl.ds(s * RPS, RPS)])


def _make_hop(width):
    @functools.partial(
        pl.kernel,
        out_type=jax.ShapeDtypeStruct((NCORE, N_PAD, width), _f32),
        mesh=_sc_mesh(),
        compiler_params=pltpu.CompilerParams(use_tc_tiling_on_sc=False),
        scratch_types=[
            pltpu.VMEM((NCHUNK, CHUNK), jnp.int32),
            pltpu.VMEM((NCHUNK, CHUNK), jnp.int32),
            pltpu.VMEM((CHUNK, width), _f32),
            pltpu.VMEM_SHARED((N_PAD, width), _f32),
            pltpu.SemaphoreType.DMA,
        ],
    )
    def hop(zs_hbm, src_hbm, dst_hbm, zero_hbm, acc_hbm,
            src_v, dst_v, gbuf, acc_sh, sem):
        c = lax.axis_index("c")
        s = lax.axis_index("s")
        w = c * NSUB + s
        pltpu.sync_copy(src_hbm.at[w], src_v)
        pltpu.sync_copy(dst_hbm.at[w], dst_v)
        pltpu.sync_copy(zero_hbm, acc_sh.at[pl.ds(s * RPS, RPS)])
        plsc.subcore_barrier()

        def body(g, carry):
            pltpu.async_copy(zs_hbm.at[src_v.at[g]], gbuf, sem).wait()
            pltpu.sync_copy(gbuf, acc_sh.at[dst_v.at[g]], add=True)
            return carry

        lax.fori_loop(0, NCHUNK, body, 0)
        plsc.subcore_barrier()
        pltpu.sync_copy(acc_sh.at[pl.ds(s * RPS, RPS)],
                        acc_hbm.at[c, pl.ds(s * RPS, RPS)])

    return hop


_hop96 = _make_hop(96)
_hop48 = _make_hop(48)


_BLK = 512


def _dense_body(x_ref, w11, w12, w13, w21, w22, w23, w31, w32, w33,
                b11, b12, b13, b21, b22, b23, b31, b32, b33,
                w1_ref, g2_ref, g3_ref, d0_ref, d1_ref,
                zs0a_ref, zs0b_ref, u1a_ref, u1b_ref, u2a_ref, u2b_ref,
                u30_ref, y1_ref, dinv_ref):
    xb = x_ref[...]
    w1b = [w1_ref[0:H, :], w1_ref[H:2 * H, :], w1_ref[2 * H:3 * H, :]]

    def t(wr, br, j):
        h = jnp.maximum(
            jnp.dot(xb, wr[...], preferred_element_type=_f32) + br[...], 0.0)
        return jnp.dot(h, w1b[j], preferred_element_type=_f32)

    t1 = [t(w11, b11, 0), t(w12, b12, 1), t(w13, b13, 2)]
    t2 = [t(w21, b21, 0), t(w22, b22, 1), t(w23, b23, 2)]
    t3 = [t(w31, b31, 0), t(w32, b32, 1), t(w33, b33, 2)]

    u2 = [g2_ref[0, k] * t2[0] + g2_ref[1, k] * t2[1] + g2_ref[2, k] * t2[2]
          for k in range(3)]
    u3 = [g3_ref[0, k] * t3[0] + g3_ref[1, k] * t3[1] + g3_ref[2, k] * t3[2]
          for k in range(4)]
    y1 = t1[0] + t1[1] + t1[2]

    deg = d0_ref[:, 0:1] + d1_ref[:, 0:1] + 1.0
    dinv = lax.rsqrt(deg)
    pad8 = jnp.zeros((xb.shape[0], 8), _f32)

    def p48(a):
        return jnp.concatenate([a, pad8], axis=1)

    dinv48 = jnp.broadcast_to(dinv, (xb.shape[0], 48))
    zs0a_ref[...] = p48(u2[2]) * dinv48
    zs0b_ref[...] = p48(u3[3]) * dinv48
    u1a_ref[...] = p48(u2[1])
    u1b_ref[...] = p48(u3[2])
    u2a_ref[...] = p48(u2[0])
    u2b_ref[...] = p48(u3[1])
    u30_ref[...] = p48(u3[0])
    y1_ref[...] = p48(y1)
    dinv_ref[...] = dinv48


def _dense_call(x_p, Ws, bs, W1, g2, g3, deg0, deg1):
    grid = (N_PAD // _BLK,)
    full = lambda shape: pl.BlockSpec(shape, lambda i: (0, 0))
    blk = lambda wdt: pl.BlockSpec((_BLK, wdt), lambda i: (i, 0))
    in_specs = (
        [blk(D)]
        + [full((D, H))] * 9
        + [full((1, H))] * 9
        + [full((3 * H, C))]
        + [pl.BlockSpec(memory_space=pltpu.SMEM)] * 2
        + [blk(16), blk(16)]
    )
    out_specs = [blk(48)] * 9
    out_shape = [jax.ShapeDtypeStruct((N_PAD, 48), _f32)] * 9
    return pl.pallas_call(
        _dense_body, grid=grid, in_specs=in_specs, out_specs=out_specs,
        out_shape=out_shape,
    )(x_p, *Ws, *[b.reshape(1, H) for b in bs], W1, g2, g3, deg0, deg1)


_BLK2 = 1024


def _comb1_body(ua_ref, ub_ref, a0a_ref, a1a_ref, a0b_ref, a1b_ref,
                zsa_ref, zsb_ref, dinv_ref, outa_ref, outb_ref):
    dinv = dinv_ref[...]
    za = ua_ref[...] + dinv * (a0a_ref[...] + a1a_ref[...] + zsa_ref[...])
    zb = ub_ref[...] + dinv * (a0b_ref[...] + a1b_ref[...] + zsb_ref[...])
    outa_ref[...] = za * dinv
    outb_ref[...] = zb * dinv


def _comb2_body(ua_ref, ub_ref, a0a_ref, a1a_ref, a0b_ref, a1b_ref,
                zsa_ref, zsb_ref, dinv_ref, r2_ref, zs3_ref):
    dinv = dinv_ref[...]
    za = ua_ref[...] + dinv * (a0a_ref[...] + a1a_ref[...] + zsa_ref[...])
    zb = ub_ref[...] + dinv * (a0b_ref[...] + a1b_ref[...] + zsb_ref[...])
    r2_ref[...] = za
    zs3_ref[...] = zb * dinv


def _comb(body, UA, UB, acc, zsA, zsB, dinv48):
    grid = (N_PAD // _BLK2,)
    blk = pl.BlockSpec((_BLK2, 48), lambda i: (i, 0))
    return pl.pallas_call(
        body, grid=grid,
        in_specs=[blk] * 9,
        out_specs=[blk, blk],
        out_shape=[jax.ShapeDtypeStruct((N_PAD, 48), _f32),
                   jax.ShapeDtypeStruct((N_PAD, 48), _f32)],
    )(UA, UB, acc[0, :, 0:48], acc[1, :, 0:48], acc[0, :, 48:96],
      acc[1, :, 48:96], zsA, zsB, dinv48)


_BLKF = 1000


def _final_body(y1_ref, r2_ref, u30_ref, a0_ref, a1_ref, zs3_ref, dinv_ref,
                b1_ref, out_ref):
    acc = (a0_ref[...] + a1_ref[...] + zs3_ref[...])[:, 0:C]
    v = (y1_ref[:, 0:C] + r2_ref[:, 0:C] + u30_ref[:, 0:C]
         + dinv_ref[:, 0:C] * acc + b1_ref[...])
    m = jnp.max(v, axis=1, keepdims=True)
    ex = jnp.exp(v - m)
    lse = jnp.log(jnp.sum(ex, axis=1, keepdims=True))
    out_ref[...] = v - m - lse


def _final(y1, r2, u30, a0, a1, zs3, dinv48, b1):
    grid = (N // _BLKF,)
    blk = lambda wdt: pl.BlockSpec((_BLKF, wdt), lambda i: (i, 0))
    return pl.pallas_call(
        _final_body, grid=grid,
        in_specs=[blk(48), blk(48), blk(48), blk(48), blk(48), blk(48),
                  blk(48), pl.BlockSpec((1, C), lambda i: (0, 0))],
        out_specs=blk(C),
        out_shape=jax.ShapeDtypeStruct((N, C), _f32),
    )(y1, r2, u30, a0, a1, zs3, dinv48, b1)


def kernel(x, edge_index, W11, b11, W12, b12, W13, b13, W21, b21, W22, b22,
           W23, b23, W31, b31, W32, b32, W33, b33, W1, b1,
           g21, g22, g23, g31, g32, g33):
    src = edge_index[0]
    dst = edge_index[1]
    flat = NW * NCHUNK * CHUNK
    srcp = jnp.concatenate([src, jnp.zeros((flat - E,), jnp.int32)]
                           ).reshape(NW, NCHUNK, CHUNK)
    dstp = jnp.concatenate([dst, jnp.full((flat - E,), N_PAD - 1, jnp.int32)]
                           ).reshape(NW, NCHUNK, CHUNK)
    x_p = jnp.pad(x, ((0, N_PAD - N), (0, 0)))

    ones16 = jnp.ones((CHUNK, 16), _f32)
    zero16 = jnp.zeros((RPS, 16), _f32)
    zero48 = jnp.zeros((RPS, 48), _f32)
    zero96 = jnp.zeros((RPS, 96), _f32)

    degacc = _deg_kernel(dstp, ones16, zero16)

    g2 = jnp.stack([g21, g22, g23])
    g3 = jnp.stack([g31, g32, g33])
    Ws = (W11, W12, W13, W21, W22, W23, W31, W32, W33)
    bs = (b11, b12, b13, b21, b22, b23, b31, b32, b33)
    zs0A, zs0B, U1A, U1B, U2A, U2B, u30, y1, dinv48 = _dense_call(
        x_p, Ws, bs, W1, g2, g3, degacc[0], degacc[1])

    zs0 = jnp.concatenate([zs0A, zs0B], axis=1)
    acc1 = _hop96(zs0, srcp, dstp, zero96)
    zs1A, zs1B = _comb(_comb1_body, U1A, U1B, acc1, zs0A, zs0B, dinv48)
    zs1 = jnp.concatenate([zs1A, zs1B], axis=1)
    acc2 = _hop96(zs1, srcp, dstp, zero96)
    r2, zs3 = _comb(_comb2_body, U2A, U2B, acc2, zs1A, zs1B, dinv48)
    acc3 = _hop48(zs3, srcp, dstp, zero48)

    return _final(y1[:N], r2[:N], u30[:N], acc3[0, :N], acc3[1, :N],
                  zs3[:N], dinv48[:N], b1.reshape(1, C))
